# baseline (speedup 1.0000x reference)
import functools

import jax
import jax.numpy as jnp
from jax import lax
from jax.experimental import pallas as pl
from jax.experimental.pallas import tpu as pltpu

B = 1
T = 2048
SPEC = 229
D = 512
H = 8
DH = 64
DEPTH = 8
BUCKET = 64
NCHUNK = T // BUCKET
NCLS = NCHUNK
NR = NCHUNK // 2
FF = 2048
OUT = 88
TILE = 256
NT = T // TILE
CPT = TILE // BUCKET
KPAD = 704

_F32 = jnp.float32


def _layer_norm(x, s, b):
    m = jnp.mean(x, axis=-1, keepdims=True)
    v = jnp.mean((x - m) ** 2, axis=-1, keepdims=True)
    return (x - m) * lax.rsqrt(v + 1e-5) * s + b


def _mm(a, b):
    return jax.lax.dot_general(a, b, (((1,), (0,)), ((), ())),
                               preferred_element_type=_F32,
                               precision=lax.Precision.HIGHEST)


def _mmbf(a, b, dims=(((1,), (0,)), ((), ()))):
    return jax.lax.dot_general(a.astype(jnp.bfloat16), b.astype(jnp.bfloat16),
                               dims, preferred_element_type=_F32)


def _front_body(xcat_ref, w_ref, b_ref, ax1_ref, ax2_ref, o_ref):
    y = _mm(xcat_ref[...], w_ref[...]) + b_ref[...]
    y = jnp.maximum(y, 0.0)
    pos = (ax1_ref[...] + ax2_ref[...]).reshape(TILE, D)
    o_ref[...] = y + pos


def _front(xcat, wflat, cb, ax1, ax2):
    return pl.pallas_call(
        _front_body,
        grid=(NT,),
        in_specs=[
            pl.BlockSpec((TILE, KPAD), lambda i: (i, 0)),
            pl.BlockSpec((KPAD, D), lambda i: (0, 0)),
            pl.BlockSpec((1, D), lambda i: (0, 0)),
            pl.BlockSpec((CPT, 1, D), lambda i: (i, 0, 0)),
            pl.BlockSpec((1, BUCKET, D), lambda i: (0, 0, 0)),
        ],
        out_specs=pl.BlockSpec((TILE, D), lambda i: (i, 0)),
        out_shape=jax.ShapeDtypeStruct((T, D), _F32),
    )(xcat, wflat, cb, ax1, ax2)


def _pre_body(y_ref, s_ref, b_ref, wqk_ref, wv_ref, rot_ref,
              qk_ref, v_ref, oh_ref, rank_ref, cnt_ref):
    i = pl.program_id(0)

    @pl.when(i == 0)
    def _():
        cnt_ref[...] = jnp.zeros_like(cnt_ref)

    xln = _layer_norm(y_ref[...], s_ref[...], b_ref[...])
    qkf = _mmbf(xln, wqk_ref[...])
    vf = _mmbf(xln, wv_ref[...])

    iota_c = lax.broadcasted_iota(jnp.int32, (TILE, NCLS), 1)
    r_iota = lax.broadcasted_iota(jnp.int32, (TILE, TILE), 0)
    c_iota = lax.broadcasted_iota(jnp.int32, (TILE, TILE), 1)
    trilstrict = jnp.where(c_iota < r_iota, 1.0, 0.0)

    for h in range(H):
        qk_h = qkf[:, h * DH:(h + 1) * DH]
        v_h = vf[:, h * DH:(h + 1) * DH]
        qk_ref[h] = qk_h
        v_ref[h] = v_h
        rot_h = _mmbf(qk_h, rot_ref[...])
        cat = jnp.concatenate([rot_h, -rot_h], axis=1)
        m = jnp.max(cat, axis=1, keepdims=True)
        bidx = jnp.min(jnp.where(cat >= m, iota_c, 2 * NCLS),
                       axis=1, keepdims=True)
        onehot = jnp.where(iota_c == bidx, 1.0, 0.0)
        oh_ref[h] = onehot
        prior = cnt_ref[h, 0, :]
        excl = _mm(trilstrict, onehot)
        rank = jnp.sum(onehot * (excl + prior[None, :]), axis=1)
        rank_ref[h, 0, :] = rank
        cnt_ref[h, 0, :] = prior + jnp.sum(onehot, axis=0)


def _pre(y, s, b, wqk, wv, rot):
    return pl.pallas_call(
        _pre_body,
        grid=(NT,),
        in_specs=[
            pl.BlockSpec((TILE, D), lambda i: (i, 0)),
            pl.BlockSpec((1, D), lambda i: (0, 0)),
            pl.BlockSpec((1, D), lambda i: (0, 0)),
            pl.BlockSpec((D, D), lambda i: (0, 0)),
            pl.BlockSpec((D, D), lambda i: (0, 0)),
            pl.BlockSpec((DH, NR), lambda i: (0, 0)),
        ],
        out_specs=[
            pl.BlockSpec((H, TILE, DH), lambda i: (0, i, 0)),
            pl.BlockSpec((H, TILE, DH), lambda i: (0, i, 0)),
            pl.BlockSpec((H, TILE, NCLS), lambda i: (0, i, 0)),
            pl.BlockSpec((H, 1, TILE), lambda i: (0, 0, i)),
            pl.BlockSpec((H, 1, NCLS), lambda i: (0, 0, 0)),
        ],
        out_shape=[
            jax.ShapeDtypeStruct((H, T, DH), _F32),
            jax.ShapeDtypeStruct((H, T, DH), _F32),
            jax.ShapeDtypeStruct((H, T, NCLS), _F32),
            jax.ShapeDtypeStruct((H, 1, T), _F32),
            jax.ShapeDtypeStruct((H, 1, NCLS), _F32),
        ],
    )(y, s, b, wqk, wv, rot)


def _perm_body(qk_ref, v_ref, oh_ref, rank_ref, cnt_ref,
               sqk_ref, sv_ref, sidx_ref, inv_ref):
    j = pl.program_id(1)
    cnt = cnt_ref[0, 0, :]
    a_i = lax.broadcasted_iota(jnp.int32, (NCLS, NCLS), 0)
    b_i = lax.broadcasted_iota(jnp.int32, (NCLS, NCLS), 1)
    ustrict = jnp.where(a_i < b_i, 1.0, 0.0)
    offsets = _mm(cnt[None, :], ustrict)[0]
    oh = oh_ref[0]
    inv = rank_ref[0, 0, :] + jnp.sum(oh * offsets[None, :], axis=1)
    oh_t = oh_ref[0, pl.ds(j * TILE, TILE), :]
    inv_ref[0, 0, :] = (rank_ref[0, 0, pl.ds(j * TILE, TILE)]
                        + jnp.sum(oh_t * offsets[None, :], axis=1))

    p_iota = (lax.broadcasted_iota(jnp.int32, (TILE, T), 0)
              + TILE * j).astype(_F32)
    ohp = jnp.where(inv[None, :] == p_iota, 1.0, 0.0)
    sqk_ref[0] = _mm(ohp, qk_ref[0])
    sv_ref[0] = _mm(ohp, v_ref[0])
    t_iota = lax.broadcasted_iota(jnp.int32, (TILE, T), 1).astype(_F32)
    sidx_ref[0, 0, :] = jnp.sum(ohp * t_iota, axis=1)


def _perm(qk, v, oh, rank, cnt):
    return pl.pallas_call(
        _perm_body,
        grid=(H, NT),
        in_specs=[
            pl.BlockSpec((1, T, DH), lambda h, j: (h, 0, 0)),
            pl.BlockSpec((1, T, DH), lambda h, j: (h, 0, 0)),
            pl.BlockSpec((1, T, NCLS), lambda h, j: (h, 0, 0)),
            pl.BlockSpec((1, 1, T), lambda h, j: (h, 0, 0)),
            pl.BlockSpec((1, 1, NCLS), lambda h, j: (h, 0, 0)),
        ],
        out_specs=[
            pl.BlockSpec((1, TILE, DH), lambda h, j: (h, j, 0)),
            pl.BlockSpec((1, TILE, DH), lambda h, j: (h, j, 0)),
            pl.BlockSpec((1, 1, TILE), lambda h, j: (h, 0, j)),
            pl.BlockSpec((1, 1, TILE), lambda h, j: (h, 0, j)),
        ],
        out_shape=[
            jax.ShapeDtypeStruct((H, T, DH), _F32),
            jax.ShapeDtypeStruct((H, T, DH), _F32),
            jax.ShapeDtypeStruct((H, 1, T), _F32),
            jax.ShapeDtypeStruct((H, 1, T), _F32),
        ],
    )(qk, v, oh, rank, cnt)


def _att_body(sqk_ref, sqkp_ref, sv_ref, svp_ref, si_ref, sip_ref, o_ref):
    scale = 1.0 / (DH ** 0.5)
    for c in range(CPT):
        q = sqk_ref[0, c * BUCKET:(c + 1) * BUCKET, :]
        qi = si_ref[0, 0, c * BUCKET:(c + 1) * BUCKET]
        if c == 0:
            kprev = sqkp_ref[0]
            vprev = svp_ref[0]
            iprev = sip_ref[0, 0, (CPT - 1) * BUCKET:]
        else:
            kprev = sqk_ref[0, (c - 1) * BUCKET:c * BUCKET, :]
            vprev = sv_ref[0, (c - 1) * BUCKET:c * BUCKET, :]
            iprev = si_ref[0, 0, (c - 1) * BUCKET:c * BUCKET]
        k_ = jnp.concatenate([q, kprev], axis=0)
        v_ = jnp.concatenate([sv_ref[0, c * BUCKET:(c + 1) * BUCKET, :], vprev],
                             axis=0)
        ki = jnp.concatenate([qi, iprev], axis=0)
        nrm = jnp.sqrt(jnp.sum(k_ * k_, axis=1, keepdims=True))
        kn = k_ / (nrm + 1e-6)
        dots = _mmbf(q, kn, (((1,), (1,)), ((), ()))) * scale
        mask = qi[:, None] == ki[None, :]
        dots = jnp.where(mask, -1e5, dots)
        m = jnp.max(dots, axis=1, keepdims=True)
        e = jnp.exp(dots - m)
        attn = e / jnp.sum(e, axis=1, keepdims=True)
        o_ref[0, c * BUCKET:(c + 1) * BUCKET, :] = _mmbf(attn, v_)


def _att(sqk, sv, sidx):
    prev = lambda h, j: (h, (j * CPT - 1) % NCHUNK, 0)
    prev2 = lambda h, j: (h, 0, (j - 1) % NT)
    return pl.pallas_call(
        _att_body,
        grid=(H, NT),
        in_specs=[
            pl.BlockSpec((1, TILE, DH), lambda h, j: (h, j, 0)),
            pl.BlockSpec((1, BUCKET, DH), prev),
            pl.BlockSpec((1, TILE, DH), lambda h, j: (h, j, 0)),
            pl.BlockSpec((1, BUCKET, DH), prev),
            pl.BlockSpec((1, 1, TILE), lambda h, j: (h, 0, j)),
            pl.BlockSpec((1, 1, TILE), prev2),
        ],
        out_specs=pl.BlockSpec((1, TILE, DH), lambda h, j: (h, j, 0)),
        out_shape=jax.ShapeDtypeStruct((H, T, DH), _F32),
    )(sqk, sqk, sv, sv, sidx, sidx)


def _post_body(y_ref, inv_ref, so_ref, wo_ref, o_ref):
    p_iota = lax.broadcasted_iota(jnp.int32, (TILE, T), 1).astype(_F32)
    outs = []
    for h in range(H):
        inv_t = inv_ref[h, 0, :]
        ohg = jnp.where(inv_t[:, None] == p_iota, 1.0, 0.0)
        outs.append(_mm(ohg, so_ref[h]))
    attn_cat = jnp.concatenate(outs, axis=1)
    o_ref[...] = y_ref[...] + _mmbf(attn_cat, wo_ref[...])


def _post(y, inv, sout, wo):
    return pl.pallas_call(
        _post_body,
        grid=(NT,),
        in_specs=[
            pl.BlockSpec((TILE, D), lambda i: (i, 0)),
            pl.BlockSpec((H, 1, TILE), lambda i: (0, 0, i)),
            pl.BlockSpec((H, T, DH), lambda i: (0, 0, 0)),
            pl.BlockSpec((D, D), lambda i: (0, 0)),
        ],
        out_specs=pl.BlockSpec((TILE, D), lambda i: (i, 0)),
        out_shape=jax.ShapeDtypeStruct((T, D), _F32),
    )(y, inv, sout, wo)


def _ffn_body(y_ref, s_ref, b_ref, w1_ref, b1_ref, w2_ref, b2_ref, o_ref):
    xln = _layer_norm(y_ref[...], s_ref[...], b_ref[...])
    hmid = jax.nn.gelu(_mmbf(xln, w1_ref[...]) + b1_ref[...])
    o_ref[...] = y_ref[...] + _mmbf(hmid, w2_ref[...]) + b2_ref[...]


def _ffn(y, s, b, w1, b1, w2, b2):
    return pl.pallas_call(
        _ffn_body,
        grid=(NT,),
        in_specs=[
            pl.BlockSpec((TILE, D), lambda i: (i, 0)),
            pl.BlockSpec((1, D), lambda i: (0, 0)),
            pl.BlockSpec((1, D), lambda i: (0, 0)),
            pl.BlockSpec((D, FF), lambda i: (0, 0)),
            pl.BlockSpec((1, FF), lambda i: (0, 0)),
            pl.BlockSpec((FF, D), lambda i: (0, 0)),
            pl.BlockSpec((1, D), lambda i: (0, 0)),
        ],
        out_specs=pl.BlockSpec((TILE, D), lambda i: (i, 0)),
        out_shape=jax.ShapeDtypeStruct((T, D), _F32),
    )(y, s, b, w1, b1, w2, b2)


def _final_body(y_ref, w_ref, b_ref, o_ref):
    o_ref[0] = _mmbf(y_ref[...], w_ref[...]) + b_ref[...]


def _final(y, w, b):
    return pl.pallas_call(
        _final_body,
        grid=(NT,),
        in_specs=[
            pl.BlockSpec((TILE, D), lambda i: (i, 0)),
            pl.BlockSpec((D, OUT), lambda i: (0, 0)),
            pl.BlockSpec((1, OUT), lambda i: (0, 0)),
        ],
        out_specs=pl.BlockSpec((1, TILE, OUT), lambda i: (0, i, 0)),
        out_shape=jax.ShapeDtypeStruct((B, T, OUT), _F32),
    )(y, w, b)


def kernel(spec, conv_w, conv_b, ax1, ax2, ln1_s, ln1_b, Wqk, Wv, Wo,
           ln2_s, ln2_b, W1, b1, W2, b2, lin_w, lin_b):
    x = spec[0]
    z = jnp.zeros((1, SPEC), _F32)
    xprev = jnp.concatenate([z, x[:-1]], axis=0)
    xnext = jnp.concatenate([x[1:], z], axis=0)
    xcat = jnp.concatenate([xprev, x, xnext], axis=1)
    xcat = jnp.pad(xcat, ((0, 0), (0, KPAD - 3 * SPEC)))
    wflat = conv_w.transpose(2, 1, 0).reshape(3 * SPEC, D)
    wflat = jnp.pad(wflat, ((0, KPAD - 3 * SPEC), (0, 0)))

    rot = jax.random.normal(jax.random.key(42), (DEPTH, DH, NR),
                            dtype=jnp.float32)

    y = _front(xcat, wflat, conv_b[None, :], ax1, ax2)

    for i in range(DEPTH):
        qk, v, oh, rank, cnt = _pre(y, ln1_s[i][None, :], ln1_b[i][None, :],
                                    Wqk[i], Wv[i], rot[i])
        sqk, sv, sidx, inv = _perm(qk, v, oh, rank, cnt)
        sout = _att(sqk, sv, sidx)
        y = _post(y, inv, sout, Wo[i])
        y = _ffn(y, ln2_s[i][None, :], ln2_b[i][None, :],
                 W1[i], b1[i][None, :], W2[i], b2[i][None, :])

    return _final(y, lin_w, lin_b[None, :])
